# baseline (speedup 1.0000x reference)
import functools

import jax
import jax.numpy as jnp
from jax import lax
from jax.experimental import pallas as pl
from jax.experimental.pallas import tpu as pltpu
from jax.experimental.pallas import tpu_sc as plsc

NC = 2
NS = 16
K = 128
CPW = 80
NW = NC * NS
EPW = K * CPW
E_PAD = NW * EPW
NP = 10240
STRIP = NP // NS
ZR = 160

_mesh = plsc.VectorSubcoreMesh(
    core_axis_name="c", subcore_axis_name="s", num_cores=NC, num_subcores=NS)
_sc_params = pltpu.CompilerParams(use_tc_tiling_on_sc=False)


def _deg_kernel():
    @functools.partial(
        pl.kernel,
        out_type=jax.ShapeDtypeStruct((NC, NP, 16), jnp.float32),
        mesh=_mesh,
        scratch_types=[
            pltpu.VMEM((CPW, K), jnp.int32),
            pltpu.VMEM((K, 16), jnp.float32),
            pltpu.VMEM((STRIP, 16), jnp.float32),
            pltpu.VMEM_SHARED((NP, 16), jnp.float32),
        ],
        compiler_params=_sc_params,
    )
    def deg(dst_hbm, ones_hbm, zeros_hbm, out_hbm, dst_v, ones_v, zbuf, acc):
        c = lax.axis_index("c")
        s = lax.axis_index("s")
        wid = s * NC + c
        pltpu.sync_copy(dst_hbm.at[pl.ds(wid * CPW, CPW)], dst_v)
        pltpu.sync_copy(ones_hbm, ones_v)
        pltpu.sync_copy(zeros_hbm.at[pl.ds(s * STRIP, STRIP)], zbuf)
        pltpu.sync_copy(zbuf, acc.at[pl.ds(s * STRIP, STRIP)])
        plsc.subcore_barrier()

        def chunk(j, carry):
            pltpu.sync_copy(ones_v, acc.at[dst_v.at[j]], add=True)
            return carry

        lax.fori_loop(0, CPW, chunk, 0)
        plsc.subcore_barrier()
        pltpu.sync_copy(acc.at[pl.ds(s * STRIP, STRIP)], zbuf)
        pltpu.sync_copy(zbuf, out_hbm.at[c, pl.ds(s * STRIP, STRIP)])

    return deg


def _agg_kernel(D, NBUF=5):
    G = CPW // NBUF

    @functools.partial(
        pl.kernel,
        out_type=jax.ShapeDtypeStruct((NC, NP, D), jnp.float32),
        mesh=_mesh,
        scratch_types=(
            [pltpu.VMEM((CPW, K), jnp.int32),
             pltpu.VMEM((CPW, K), jnp.int32)]
            + [pltpu.VMEM((K, D), jnp.float32)] * NBUF
            + [pltpu.VMEM((ZR, D), jnp.float32),
               pltpu.VMEM_SHARED((NP, D), jnp.float32)]
            + [pltpu.SemaphoreType.DMA] * NBUF
        ),
        compiler_params=_sc_params,
    )
    def agg(src_hbm, dst_hbm, table_hbm, zeros_hbm, out_hbm,
            src_v, dst_v, *rest):
        rows = rest[:NBUF]
        zbuf = rest[NBUF]
        acc = rest[NBUF + 1]
        sems = rest[NBUF + 2:]
        c = lax.axis_index("c")
        s = lax.axis_index("s")
        wid = s * NC + c
        with jax.named_scope("agg_init"):
            pltpu.sync_copy(src_hbm.at[pl.ds(wid * CPW, CPW)], src_v)
            pltpu.sync_copy(dst_hbm.at[pl.ds(wid * CPW, CPW)], dst_v)
            pltpu.sync_copy(zeros_hbm, zbuf)
            for t in range(STRIP // ZR):
                pltpu.sync_copy(zbuf, acc.at[pl.ds(s * STRIP + t * ZR, ZR)])
            plsc.subcore_barrier()

        def group(g, carry):
            hs = [pltpu.async_copy(table_hbm.at[src_v.at[g * NBUF + b]],
                                   rows[b], sems[b])
                  for b in range(NBUF)]
            for b in range(NBUF):
                hs[b].wait()
                pltpu.sync_copy(rows[b], acc.at[dst_v.at[g * NBUF + b]],
                                add=True)
            return carry

        with jax.named_scope("agg_gather"):
            lax.fori_loop(0, G, group, 0)
            plsc.subcore_barrier()
        with jax.named_scope("agg_flush"):
            for t in range(STRIP // ZR):
                pltpu.sync_copy(acc.at[pl.ds(s * STRIP + t * ZR, ZR)], zbuf)
                pltpu.sync_copy(zbuf, out_hbm.at[c, pl.ds(s * STRIP + t * ZR, ZR)])

    return agg


def _agg_fs_kernel(DH, NBUF=4):
    CPT = E_PAD // (NS * K)
    G = CPT // NBUF

    @functools.partial(
        pl.kernel,
        out_type=jax.ShapeDtypeStruct((NC, NP, DH), jnp.float32),
        mesh=_mesh,
        scratch_types=(
            [pltpu.VMEM((CPT, K), jnp.int32),
             pltpu.VMEM((CPT, K), jnp.int32)]
            + [pltpu.VMEM((K, DH), jnp.float32)] * NBUF
            + [pltpu.VMEM((ZR, DH), jnp.float32),
               pltpu.VMEM_SHARED((NP, DH), jnp.float32),
               pltpu.VMEM_SHARED((NP, DH), jnp.float32)]
            + [pltpu.SemaphoreType.DMA] * NBUF
        ),
        compiler_params=_sc_params,
    )
    def agg(src_hbm, dst_hbm, table_hbm, zeros_hbm, out_hbm,
            src_v, dst_v, *rest):
        rows = rest[:NBUF]
        zbuf = rest[NBUF]
        tbl = rest[NBUF + 1]
        acc = rest[NBUF + 2]
        sems = rest[NBUF + 3:]
        c = lax.axis_index("c")
        s = lax.axis_index("s")
        with jax.named_scope("agg_init"):
            pltpu.sync_copy(src_hbm.at[pl.ds(s * CPT, CPT)], src_v)
            pltpu.sync_copy(dst_hbm.at[pl.ds(s * CPT, CPT)], dst_v)
            pltpu.sync_copy(table_hbm.at[c, pl.ds(s * STRIP, STRIP)],
                            tbl.at[pl.ds(s * STRIP, STRIP)])
            pltpu.sync_copy(zeros_hbm, zbuf)
            for t in range(STRIP // ZR):
                pltpu.sync_copy(zbuf, acc.at[pl.ds(s * STRIP + t * ZR, ZR)])
            plsc.subcore_barrier()

        def group(g, carry):
            hs = [pltpu.async_copy(tbl.at[src_v.at[g * NBUF + b]],
                                   rows[b], sems[b])
                  for b in range(NBUF)]
            for b in range(NBUF):
                hs[b].wait()
                pltpu.sync_copy(rows[b], acc.at[dst_v.at[g * NBUF + b]],
                                add=True)
            return carry

        with jax.named_scope("agg_gather"):
            lax.fori_loop(0, G, group, 0)
            plsc.subcore_barrier()
        with jax.named_scope("agg_flush"):
            for t in range(STRIP // ZR):
                pltpu.sync_copy(acc.at[pl.ds(s * STRIP + t * ZR, ZR)], zbuf)
                pltpu.sync_copy(zbuf, out_hbm.at[c, pl.ds(s * STRIP + t * ZR, ZR)])

    return agg


_deg = _deg_kernel()
_agg64 = _agg_kernel(64)
_aggfs = _agg_fs_kernel(32)


def _tc_enc(x_ref, w_ref, degp_ref, out_ref, dis_ref):
    counts = degp_ref[0, :, 0:1] + degp_ref[1, :, 0:1]
    dis = lax.rsqrt(counts + 1.0)
    dis_ref[...] = dis
    h = jnp.dot(x_ref[...], w_ref[...], preferred_element_type=jnp.float32)
    out_ref[...] = h * dis


def _tc_mid(aggp_ref, h1p_ref, dis_ref, b_ref, out_ref):
    dis = dis_ref[...]
    z = (aggp_ref[0] + aggp_ref[1] + h1p_ref[...]) * dis + b_ref[...]
    out_ref[...] = z * dis


def _tc_out(aggp_ref, zp_ref, dis_ref, b_ref, w_ref, out_ref):
    t = aggp_ref[0] + aggp_ref[1] + zp_ref[...]
    h2 = jnp.dot(t, w_ref[...], preferred_element_type=jnp.float32)
    out_ref[...] = h2 * dis_ref[...] + b_ref[...]


def kernel(x, edge_index, W_enc, b_enc, W_dec, b_dec):
    N, D_IN = x.shape
    D_BOT = W_enc.shape[1]
    E = edge_index.shape[1]

    ar = jnp.arange(E_PAD - E, dtype=edge_index.dtype)
    pad_src = ar % NP
    pad_dst = N + ar % (NP - N)
    src = jnp.concatenate([edge_index[0], pad_src]).reshape(E_PAD // K, K)
    dst = jnp.concatenate([edge_index[1], pad_dst]).reshape(E_PAD // K, K)
    x_pad = jnp.pad(x, ((0, NP - N), (0, 0)))

    ones1 = jnp.ones((K, 16), jnp.float32)
    zeros1 = jnp.zeros((NP, 16), jnp.float32)
    zeros_bot = jnp.zeros((ZR, D_BOT), jnp.float32)

    degp = _deg(dst, ones1, zeros1)

    h1p, dis = pl.pallas_call(
        _tc_enc,
        out_shape=(jax.ShapeDtypeStruct((NP, D_BOT), jnp.float32),
                   jax.ShapeDtypeStruct((NP, 1), jnp.float32)),
    )(x_pad, W_enc, degp)

    agg1 = _agg64(src, dst, h1p, zeros_bot)

    zp = pl.pallas_call(
        _tc_mid,
        out_shape=jax.ShapeDtypeStruct((NP, D_BOT), jnp.float32),
    )(agg1, h1p, dis, b_enc.reshape(1, D_BOT))

    agg2 = _agg64(src, dst, zp, zeros_bot)

    out = pl.pallas_call(
        _tc_out,
        out_shape=jax.ShapeDtypeStruct((NP, D_IN), jnp.float32),
    )(agg2, zp, dis, b_dec.reshape(1, D_IN), W_dec)

    return out[:N]


# speedup vs baseline: 1.0411x; 1.0411x over previous
import functools

import jax
import jax.numpy as jnp
from jax import lax
from jax.experimental import pallas as pl
from jax.experimental.pallas import tpu as pltpu
from jax.experimental.pallas import tpu_sc as plsc

NC = 2
NS = 16
K = 128
CPW = 80
NW = NC * NS
EPW = K * CPW
E_PAD = NW * EPW
NP = 10240
STRIP = NP // NS
ZR = 160

_mesh = plsc.VectorSubcoreMesh(
    core_axis_name="c", subcore_axis_name="s", num_cores=NC, num_subcores=NS)
_sc_params = pltpu.CompilerParams(use_tc_tiling_on_sc=False)


def _deg_kernel():
    @functools.partial(
        pl.kernel,
        out_type=jax.ShapeDtypeStruct((NC, NP, 16), jnp.float32),
        mesh=_mesh,
        scratch_types=[
            pltpu.VMEM((CPW, K), jnp.int32),
            pltpu.VMEM((K, 16), jnp.float32),
            pltpu.VMEM((STRIP, 16), jnp.float32),
            pltpu.VMEM_SHARED((NP, 16), jnp.float32),
            pltpu.SemaphoreType.DMA,
        ],
        compiler_params=_sc_params,
    )
    def deg(dst_hbm, ones_hbm, zeros_hbm, out_hbm, dst_v, ones_v, zbuf, acc,
            sem):
        c = lax.axis_index("c")
        s = lax.axis_index("s")
        wid = s * NC + c
        pltpu.sync_copy(dst_hbm.at[pl.ds(wid * CPW, CPW)], dst_v)
        pltpu.sync_copy(ones_hbm, ones_v)
        pltpu.sync_copy(zeros_hbm.at[pl.ds(s * STRIP, STRIP)], zbuf)
        pltpu.sync_copy(zbuf, acc.at[pl.ds(s * STRIP, STRIP)])
        plsc.subcore_barrier()

        def group(g, carry):
            hs = [pltpu.async_copy(ones_v, acc.at[dst_v.at[g * 8 + b]],
                                   sem, add=True)
                  for b in range(8)]
            for h in hs:
                h.wait()
            return carry

        lax.fori_loop(0, CPW // 8, group, 0)
        plsc.subcore_barrier()
        pltpu.sync_copy(acc.at[pl.ds(s * STRIP, STRIP)], zbuf)
        pltpu.sync_copy(zbuf, out_hbm.at[c, pl.ds(s * STRIP, STRIP)])

    return deg


def _agg_kernel(D, NBUF=5):
    G = CPW // NBUF

    @functools.partial(
        pl.kernel,
        out_type=jax.ShapeDtypeStruct((NC, NP, D), jnp.float32),
        mesh=_mesh,
        scratch_types=(
            [pltpu.VMEM((CPW, K), jnp.int32),
             pltpu.VMEM((CPW, K), jnp.int32)]
            + [pltpu.VMEM((K, D), jnp.float32)] * NBUF
            + [pltpu.VMEM((ZR, D), jnp.float32),
               pltpu.VMEM_SHARED((NP, D), jnp.float32)]
            + [pltpu.SemaphoreType.DMA] * (2 * NBUF)
        ),
        compiler_params=_sc_params,
    )
    def agg(src_hbm, dst_hbm, table_hbm, zeros_hbm, out_hbm,
            src_v, dst_v, *rest):
        rows = rest[:NBUF]
        zbuf = rest[NBUF]
        acc = rest[NBUF + 1]
        sems = rest[NBUF + 2:NBUF + 2 + NBUF]
        ssems = rest[NBUF + 2 + NBUF:]
        c = lax.axis_index("c")
        s = lax.axis_index("s")
        wid = s * NC + c
        with jax.named_scope("agg_init"):
            pltpu.sync_copy(src_hbm.at[pl.ds(wid * CPW, CPW)], src_v)
            pltpu.sync_copy(dst_hbm.at[pl.ds(wid * CPW, CPW)], dst_v)
            pltpu.sync_copy(zeros_hbm, zbuf)
            for t in range(STRIP // ZR):
                pltpu.sync_copy(zbuf, acc.at[pl.ds(s * STRIP + t * ZR, ZR)])
            plsc.subcore_barrier()

        def group(g, carry):
            hs = [pltpu.async_copy(table_hbm.at[src_v.at[g * NBUF + b]],
                                   rows[b], sems[b])
                  for b in range(NBUF)]
            ss = []
            for b in range(NBUF):
                hs[b].wait()
                ss.append(pltpu.async_copy(rows[b],
                                           acc.at[dst_v.at[g * NBUF + b]],
                                           ssems[b], add=True))
            for h in ss:
                h.wait()
            return carry

        with jax.named_scope("agg_gather"):
            lax.fori_loop(0, G, group, 0)
            plsc.subcore_barrier()
        with jax.named_scope("agg_flush"):
            for t in range(STRIP // ZR):
                pltpu.sync_copy(acc.at[pl.ds(s * STRIP + t * ZR, ZR)], zbuf)
                pltpu.sync_copy(zbuf, out_hbm.at[c, pl.ds(s * STRIP + t * ZR, ZR)])

    return agg


def _agg_fs_kernel(DH, NBUF=4):
    CPT = E_PAD // (NS * K)
    G = CPT // NBUF

    @functools.partial(
        pl.kernel,
        out_type=jax.ShapeDtypeStruct((NC, NP, DH), jnp.float32),
        mesh=_mesh,
        scratch_types=(
            [pltpu.VMEM((CPT, K), jnp.int32),
             pltpu.VMEM((CPT, K), jnp.int32)]
            + [pltpu.VMEM((K, DH), jnp.float32)] * NBUF
            + [pltpu.VMEM((ZR, DH), jnp.float32),
               pltpu.VMEM_SHARED((NP, DH), jnp.float32),
               pltpu.VMEM_SHARED((NP, DH), jnp.float32)]
            + [pltpu.SemaphoreType.DMA] * NBUF
        ),
        compiler_params=_sc_params,
    )
    def agg(src_hbm, dst_hbm, table_hbm, zeros_hbm, out_hbm,
            src_v, dst_v, *rest):
        rows = rest[:NBUF]
        zbuf = rest[NBUF]
        tbl = rest[NBUF + 1]
        acc = rest[NBUF + 2]
        sems = rest[NBUF + 3:]
        c = lax.axis_index("c")
        s = lax.axis_index("s")
        with jax.named_scope("agg_init"):
            pltpu.sync_copy(src_hbm.at[pl.ds(s * CPT, CPT)], src_v)
            pltpu.sync_copy(dst_hbm.at[pl.ds(s * CPT, CPT)], dst_v)
            pltpu.sync_copy(table_hbm.at[c, pl.ds(s * STRIP, STRIP)],
                            tbl.at[pl.ds(s * STRIP, STRIP)])
            pltpu.sync_copy(zeros_hbm, zbuf)
            for t in range(STRIP // ZR):
                pltpu.sync_copy(zbuf, acc.at[pl.ds(s * STRIP + t * ZR, ZR)])
            plsc.subcore_barrier()

        def group(g, carry):
            hs = [pltpu.async_copy(tbl.at[src_v.at[g * NBUF + b]],
                                   rows[b], sems[b])
                  for b in range(NBUF)]
            for b in range(NBUF):
                hs[b].wait()
                pltpu.sync_copy(rows[b], acc.at[dst_v.at[g * NBUF + b]],
                                add=True)
            return carry

        with jax.named_scope("agg_gather"):
            lax.fori_loop(0, G, group, 0)
            plsc.subcore_barrier()
        with jax.named_scope("agg_flush"):
            for t in range(STRIP // ZR):
                pltpu.sync_copy(acc.at[pl.ds(s * STRIP + t * ZR, ZR)], zbuf)
                pltpu.sync_copy(zbuf, out_hbm.at[c, pl.ds(s * STRIP + t * ZR, ZR)])

    return agg


_deg = _deg_kernel()
_agg64 = _agg_kernel(64)
_aggfs = _agg_fs_kernel(32)


def _tc_enc(x_ref, w_ref, degp_ref, out_ref, dis_ref):
    counts = degp_ref[0, :, 0:1] + degp_ref[1, :, 0:1]
    dis = lax.rsqrt(counts + 1.0)
    dis_ref[...] = dis
    h = jnp.dot(x_ref[...], w_ref[...], preferred_element_type=jnp.float32)
    out_ref[...] = h * dis


def _tc_mid(aggp_ref, h1p_ref, dis_ref, b_ref, out_ref):
    dis = dis_ref[...]
    z = (aggp_ref[0] + aggp_ref[1] + h1p_ref[...]) * dis + b_ref[...]
    out_ref[...] = z * dis


def _tc_out(aggp_ref, zp_ref, dis_ref, b_ref, w_ref, out_ref):
    t = aggp_ref[0] + aggp_ref[1] + zp_ref[...]
    h2 = jnp.dot(t, w_ref[...], preferred_element_type=jnp.float32)
    out_ref[...] = h2 * dis_ref[...] + b_ref[...]


def kernel(x, edge_index, W_enc, b_enc, W_dec, b_dec):
    N, D_IN = x.shape
    D_BOT = W_enc.shape[1]
    E = edge_index.shape[1]

    ar = jnp.arange(E_PAD - E, dtype=edge_index.dtype)
    pad_src = ar % NP
    pad_dst = N + ar % (NP - N)
    src = jnp.concatenate([edge_index[0], pad_src]).reshape(E_PAD // K, K)
    dst = jnp.concatenate([edge_index[1], pad_dst]).reshape(E_PAD // K, K)
    x_pad = jnp.pad(x, ((0, NP - N), (0, 0)))

    ones1 = jnp.ones((K, 16), jnp.float32)
    zeros1 = jnp.zeros((NP, 16), jnp.float32)
    zeros_bot = jnp.zeros((ZR, D_BOT), jnp.float32)

    degp = _deg(dst, ones1, zeros1)

    h1p, dis = pl.pallas_call(
        _tc_enc,
        out_shape=(jax.ShapeDtypeStruct((NP, D_BOT), jnp.float32),
                   jax.ShapeDtypeStruct((NP, 1), jnp.float32)),
    )(x_pad, W_enc, degp)

    agg1 = _agg64(src, dst, h1p, zeros_bot)

    zp = pl.pallas_call(
        _tc_mid,
        out_shape=jax.ShapeDtypeStruct((NP, D_BOT), jnp.float32),
    )(agg1, h1p, dis, b_enc.reshape(1, D_BOT))

    agg2 = _agg64(src, dst, zp, zeros_bot)

    out = pl.pallas_call(
        _tc_out,
        out_shape=jax.ShapeDtypeStruct((NP, D_IN), jnp.float32),
    )(agg2, zp, dis, b_dec.reshape(1, D_IN), W_dec)

    return out[:N]


# speedup vs baseline: 1.0419x; 1.0007x over previous
import functools

import jax
import jax.numpy as jnp
from jax import lax
from jax.experimental import pallas as pl
from jax.experimental.pallas import tpu as pltpu
from jax.experimental.pallas import tpu_sc as plsc

NC = 2
NS = 16
K = 128
CPW = 80
NW = NC * NS
EPW = K * CPW
E_PAD = NW * EPW
NP = 10240
STRIP = NP // NS
ZR = 160

_mesh = plsc.VectorSubcoreMesh(
    core_axis_name="c", subcore_axis_name="s", num_cores=NC, num_subcores=NS)
_sc_params = pltpu.CompilerParams(use_tc_tiling_on_sc=False)


def _deg_kernel():
    @functools.partial(
        pl.kernel,
        out_type=jax.ShapeDtypeStruct((NC, NP, 16), jnp.float32),
        mesh=_mesh,
        scratch_types=[
            pltpu.VMEM((CPW, K), jnp.int32),
            pltpu.VMEM((K, 16), jnp.float32),
            pltpu.VMEM((STRIP, 16), jnp.float32),
            pltpu.VMEM_SHARED((NP, 16), jnp.float32),
            pltpu.SemaphoreType.DMA,
        ],
        compiler_params=_sc_params,
    )
    def deg(dst_hbm, ones_hbm, zeros_hbm, out_hbm, dst_v, ones_v, zbuf, acc,
            sem):
        c = lax.axis_index("c")
        s = lax.axis_index("s")
        wid = s * NC + c
        pltpu.sync_copy(dst_hbm.at[pl.ds(wid * CPW, CPW)], dst_v)
        pltpu.sync_copy(ones_hbm, ones_v)
        pltpu.sync_copy(zeros_hbm.at[pl.ds(s * STRIP, STRIP)], zbuf)
        pltpu.sync_copy(zbuf, acc.at[pl.ds(s * STRIP, STRIP)])
        plsc.subcore_barrier()

        def group(g, carry):
            hs = [pltpu.async_copy(ones_v, acc.at[dst_v.at[g * 8 + b]],
                                   sem, add=True)
                  for b in range(8)]
            for h in hs:
                h.wait()
            return carry

        lax.fori_loop(0, CPW // 8, group, 0)
        plsc.subcore_barrier()
        pltpu.sync_copy(acc.at[pl.ds(s * STRIP, STRIP)], zbuf)
        pltpu.sync_copy(zbuf, out_hbm.at[c, pl.ds(s * STRIP, STRIP)])

    return deg


def _agg_kernel(D, NBUF=5):
    G = CPW // NBUF

    @functools.partial(
        pl.kernel,
        out_type=jax.ShapeDtypeStruct((NC, NP, D), jnp.float32),
        mesh=_mesh,
        scratch_types=(
            [pltpu.VMEM((CPW, K), jnp.int32),
             pltpu.VMEM((CPW, K), jnp.int32)]
            + [pltpu.VMEM((K, D), jnp.float32)] * NBUF
            + [pltpu.VMEM((ZR, D), jnp.float32),
               pltpu.VMEM_SHARED((NP, D), jnp.float32)]
            + [pltpu.SemaphoreType.DMA] * (2 * NBUF)
        ),
        compiler_params=_sc_params,
    )
    def agg(src_hbm, dst_hbm, table_hbm, zeros_hbm, out_hbm,
            src_v, dst_v, *rest):
        rows = rest[:NBUF]
        zbuf = rest[NBUF]
        acc = rest[NBUF + 1]
        sems = rest[NBUF + 2:NBUF + 2 + NBUF]
        ssems = rest[NBUF + 2 + NBUF:]
        c = lax.axis_index("c")
        s = lax.axis_index("s")
        wid = s * NC + c
        with jax.named_scope("agg_init"):
            pltpu.sync_copy(src_hbm.at[pl.ds(wid * CPW, CPW)], src_v)
            pltpu.sync_copy(dst_hbm.at[pl.ds(wid * CPW, CPW)], dst_v)
            pltpu.sync_copy(zeros_hbm, zbuf)
            for t in range(STRIP // ZR):
                pltpu.sync_copy(zbuf, acc.at[pl.ds(s * STRIP + t * ZR, ZR)])
            plsc.subcore_barrier()

        def group(g, carry):
            hs = [pltpu.async_copy(table_hbm.at[src_v.at[g * NBUF + b]],
                                   rows[b], sems[b])
                  for b in range(NBUF)]
            ss = []
            for b in range(NBUF):
                hs[b].wait()
                ss.append(pltpu.async_copy(rows[b],
                                           acc.at[dst_v.at[g * NBUF + b]],
                                           ssems[b], add=True))
            for h in ss:
                h.wait()
            return carry

        with jax.named_scope("agg_gather"):
            lax.fori_loop(0, G, group, 0)
            plsc.subcore_barrier()
        with jax.named_scope("agg_flush"):
            for t in range(STRIP // ZR):
                pltpu.sync_copy(acc.at[pl.ds(s * STRIP + t * ZR, ZR)], zbuf)
                pltpu.sync_copy(zbuf, out_hbm.at[c, pl.ds(s * STRIP + t * ZR, ZR)])

    return agg


_deg = _deg_kernel()
_agg64 = _agg_kernel(64)


def _tc_enc(x_ref, w_ref, degp_ref, out_ref, dis_ref):
    counts = degp_ref[0, :, 0:1] + degp_ref[1, :, 0:1]
    dis = lax.rsqrt(counts + 1.0)
    dis_ref[...] = dis
    h = jnp.dot(x_ref[...], w_ref[...], preferred_element_type=jnp.float32)
    out_ref[...] = h * dis


def _tc_mid(aggp_ref, h1p_ref, dis_ref, b_ref, out_ref):
    dis = dis_ref[...]
    z = (aggp_ref[0] + aggp_ref[1] + h1p_ref[...]) * dis + b_ref[...]
    out_ref[...] = z * dis


def _tc_out(aggp_ref, zp_ref, dis_ref, b_ref, w_ref, out_ref):
    t = aggp_ref[0] + aggp_ref[1] + zp_ref[...]
    h2 = jnp.dot(t, w_ref[...], preferred_element_type=jnp.float32)
    out_ref[...] = h2 * dis_ref[...] + b_ref[...]


def kernel(x, edge_index, W_enc, b_enc, W_dec, b_dec):
    N, D_IN = x.shape
    D_BOT = W_enc.shape[1]
    E = edge_index.shape[1]

    ar = jnp.arange(E_PAD - E, dtype=edge_index.dtype)
    pad_src = ar % NP
    pad_dst = N + ar % (NP - N)
    src = jnp.concatenate([edge_index[0], pad_src]).reshape(E_PAD // K, K)
    dst = jnp.concatenate([edge_index[1], pad_dst]).reshape(E_PAD // K, K)
    x_pad = jnp.pad(x, ((0, NP - N), (0, 0)))

    ones1 = jnp.ones((K, 16), jnp.float32)
    zeros1 = jnp.zeros((NP, 16), jnp.float32)
    zeros_bot = jnp.zeros((ZR, D_BOT), jnp.float32)

    degp = _deg(dst, ones1, zeros1)

    h1p, dis = pl.pallas_call(
        _tc_enc,
        out_shape=(jax.ShapeDtypeStruct((NP, D_BOT), jnp.float32),
                   jax.ShapeDtypeStruct((NP, 1), jnp.float32)),
    )(x_pad, W_enc, degp)

    agg1 = _agg64(src, dst, h1p, zeros_bot)

    zp = pl.pallas_call(
        _tc_mid,
        out_shape=jax.ShapeDtypeStruct((NP, D_BOT), jnp.float32),
    )(agg1, h1p, dis, b_enc.reshape(1, D_BOT))

    agg2 = _agg64(src, dst, zp, zeros_bot)

    out = pl.pallas_call(
        _tc_out,
        out_shape=jax.ShapeDtypeStruct((NP, D_IN), jnp.float32),
    )(agg2, zp, dis, b_dec.reshape(1, D_IN), W_dec)

    return out[:N]
